# baseline (speedup 1.0000x reference)
import functools
import math

import jax
import jax.numpy as jnp
from jax import lax
from jax.experimental import pallas as pl
from jax.experimental.pallas import tpu as pltpu
from jax.experimental.pallas import tpu_sc as plsc


@functools.lru_cache(maxsize=None)
def _make_sc_hist(NN: int, C: int):
    L = 16
    NW = 32
    CH = (NN // NW) // L * L
    assert CH % 8 == 0
    TAIL = NN - CH * NW
    assert TAIL % L == 0 and (CH * NW) % 8 == 0

    mesh = plsc.VectorSubcoreMesh(core_axis_name="c", subcore_axis_name="s")

    @functools.partial(
        pl.kernel,
        mesh=mesh,
        out_type=jax.ShapeDtypeStruct((NW, C), jnp.int32),
        scratch_types=[
            pltpu.VMEM((CH,), jnp.int32),
            pltpu.VMEM((L * C,), jnp.int32),
            pltpu.VMEM((C,), jnp.int32),
            pltpu.VMEM((max(TAIL, L),), jnp.int32),
        ],
        compiler_params=pltpu.CompilerParams(needs_layout_passes=False),
    )
    def sc_hist(idx_hbm, out_hbm, idx_v, hist_v, part_v, tail_v):
        wid = lax.axis_index("s") * 2 + lax.axis_index("c")
        base = wid * CH
        pltpu.sync_copy(idx_hbm.at[pl.ds(base, CH)], idx_v)

        lane_base = lax.iota(jnp.int32, 16) * C
        ones = jnp.ones((L,), jnp.int32)
        zeros = jnp.zeros((L,), jnp.int32)

        def zbody(j, _):
            hist_v[pl.ds(j * L, L)] = zeros
            return _
        lax.fori_loop(0, (L * C) // L, zbody, 0, unroll=8)

        def sbody(i, _):
            vals = idx_v[pl.ds(i * L, L)]
            plsc.addupdate_scatter(hist_v, [lane_base + vals], ones)
            return _
        lax.fori_loop(0, CH // L, sbody, 0, unroll=8)

        if TAIL > 0:
            @pl.when(wid == 0)
            def _tail():
                pltpu.sync_copy(idx_hbm.at[pl.ds(CH * NW, TAIL)], tail_v)
                def tbody(i, _):
                    vals = tail_v[pl.ds(i * L, L)]
                    plsc.addupdate_scatter(hist_v, [lane_base + vals], ones)
                    return _
                lax.fori_loop(0, TAIL // L, tbody, 0, unroll=4)

        def rbody(j, _):
            acc = hist_v[pl.ds(j * L, L)]
            for l in range(1, L):
                acc = acc + hist_v[pl.ds(j * L + l * C, L)]
            part_v[pl.ds(j * L, L)] = acc
            return _
        lax.fori_loop(0, C // L, rbody, 0, unroll=4)

        pltpu.sync_copy(part_v, out_hbm.at[wid])

    return sc_hist


def _tc_a_body(x_ref, pe_ref,
               w1_ref, b1_ref, w2_ref, b2_ref,
               proj_w_ref, proj_b_ref, qg_w_ref, qg_b_ref,
               kg_w_ref, kg_b_ref, vg_w_ref, vg_b_ref,
               vqk_ref, vqv_ref,
               q_ref, k_ref, v_ref):
    i = pl.program_id(0)
    H = q_ref.shape[1]
    scale = 1.0 / math.sqrt(H)

    @pl.when(i == 0)
    def _kv():
        k_ref[...] = (vqk_ref[...] @ kg_w_ref[...] + kg_b_ref[...]).astype(jnp.bfloat16)
        v_ref[...] = (vqv_ref[...] @ vg_w_ref[...] + vg_b_ref[...]).astype(jnp.bfloat16)

    xb = x_ref[...]
    h = jnp.maximum(xb @ w1_ref[...] + b1_ref[...], 0.0) @ w2_ref[...] + b2_ref[...]
    p = h @ proj_w_ref[...] + proj_b_ref[...]
    GD = p.shape[1]
    q = (p @ qg_w_ref[:GD, :] + pe_ref[...] @ qg_w_ref[GD:, :]) + qg_b_ref[...]
    q_ref[...] = (q * scale).astype(jnp.bfloat16)


def _tc_b_body(q_ref, cnt_ref, k_ref, v_ref,
               ff_w1_ref, ff_b1_ref, ff_w2_ref, ff_b2_ref,
               out_w_ref, out_b_ref,
               y_ref, bias_scr):
    i = pl.program_id(0)

    @pl.when(i == 0)
    def _bias():
        cnt = jnp.sum(cnt_ref[...], axis=0).astype(jnp.float32)
        bias_scr[...] = jnp.log(cnt)[None, :]

    dots = lax.dot_general(q_ref[...], k_ref[...], (((1,), (1,)), ((), ())),
                           preferred_element_type=jnp.float32)
    t = dots + bias_scr[...]
    m = jnp.max(t, axis=1, keepdims=True)
    e = jnp.exp(t - m)
    num = lax.dot_general(e.astype(jnp.bfloat16), v_ref[...],
                          (((1,), (0,)), ((), ())),
                          preferred_element_type=jnp.float32)
    out = num / jnp.sum(e, axis=1, keepdims=True)
    f = jnp.maximum(out @ ff_w1_ref[...] + ff_b1_ref[...], 0.0)
    f = jnp.maximum(f @ ff_w2_ref[...] + ff_b2_ref[...], 0.0)
    y_ref[...] = f @ out_w_ref[...] + out_b_ref[...]


def _row2(BLK, d):
    return pl.BlockSpec((BLK, d), lambda i: (i, 0))


def _full(s):
    return pl.BlockSpec(s, lambda i: (0,) * len(s))


def _vec(d):
    return pl.BlockSpec((1, d), lambda i: (0, 0))


def _tc_stage_a(x, pos_enc, params, BLK):
    B, IN = x.shape
    GD = pos_enc.shape[1]
    C = params["vq_k"].shape[0]
    H = params["qg_b"].shape[0]
    grid = (B // BLK,)

    in_specs = [
        _row2(BLK, IN), _row2(BLK, GD),
        _full((IN, H)), _vec(H),
        _full((H, H)), _vec(H),
        _full((H, GD)), _vec(GD),
        _full((2 * GD, H)), _vec(H),
        _full((2 * GD, H)), _vec(H),
        _full((GD, H)), _vec(H),
        _full((C, 2 * GD)), _full((C, GD)),
    ]
    out_specs = [_row2(BLK, H), _full((C, H)), _full((C, H))]
    out_shape = [
        jax.ShapeDtypeStruct((B, H), jnp.bfloat16),
        jax.ShapeDtypeStruct((C, H), jnp.bfloat16),
        jax.ShapeDtypeStruct((C, H), jnp.bfloat16),
    ]
    args = [
        x, pos_enc,
        params["fc_in_w1"], params["fc_in_b1"].reshape(1, -1),
        params["fc_in_w2"], params["fc_in_b2"].reshape(1, -1),
        params["proj_w"], params["proj_b"].reshape(1, -1),
        params["qg_w"], params["qg_b"].reshape(1, -1),
        params["kg_w"], params["kg_b"].reshape(1, -1),
        params["vg_w"], params["vg_b"].reshape(1, -1),
        params["vq_k"], params["vq_v"],
    ]
    return pl.pallas_call(
        _tc_a_body,
        grid=grid,
        in_specs=in_specs,
        out_specs=out_specs,
        out_shape=out_shape,
        compiler_params=pltpu.CompilerParams(
            dimension_semantics=("arbitrary",),
        ),
    )(*args)


def _tc_stage_b(q, counts_parts, k, v, params, BLK):
    B, H = q.shape
    C = k.shape[0]
    grid = (B // BLK,)

    in_specs = [
        _row2(BLK, H),
        _full(counts_parts.shape),
        _full((C, H)), _full((C, H)),
        _full((H, H)), _vec(H),
        _full((H, H)), _vec(H),
        _full((H, H)), _vec(H),
    ]
    args = [
        q, counts_parts, k, v,
        params["ff_w1"], params["ff_b1"].reshape(1, -1),
        params["ff_w2"], params["ff_b2"].reshape(1, -1),
        params["out_w"], params["out_b"].reshape(1, -1),
    ]
    return pl.pallas_call(
        _tc_b_body,
        grid=grid,
        in_specs=in_specs,
        out_specs=_row2(BLK, H),
        out_shape=jax.ShapeDtypeStruct((B, H), jnp.float32),
        scratch_shapes=[pltpu.VMEM((1, C), jnp.float32)],
        compiler_params=pltpu.CompilerParams(
            dimension_semantics=("arbitrary",),
        ),
    )(*args)


def kernel(seq, x, pos_enc, batch_idx, c_idx,
           fc_in_w1, fc_in_b1, fc_in_w2, fc_in_b2,
           fcs_w1, fcs_b1, fcs_w2, fcs_b2,
           proj_w, proj_b, qg_w, qg_b, kg_w, kg_b, vg_w, vg_b,
           vq_k, vq_v, ff_w1, ff_b1, ff_w2, ff_b2, out_w, out_b):
    NN = c_idx.shape[0]
    C = vq_k.shape[0]
    counts_parts = _make_sc_hist(NN, C)(c_idx.astype(jnp.int32))
    params = dict(
        fc_in_w1=fc_in_w1, fc_in_b1=fc_in_b1,
        fc_in_w2=fc_in_w2, fc_in_b2=fc_in_b2,
        proj_w=proj_w, proj_b=proj_b,
        qg_w=qg_w, qg_b=qg_b, kg_w=kg_w, kg_b=kg_b,
        vg_w=vg_w, vg_b=vg_b, vq_k=vq_k, vq_v=vq_v,
        ff_w1=ff_w1, ff_b1=ff_b1, ff_w2=ff_w2, ff_b2=ff_b2,
        out_w=out_w, out_b=out_b,
    )
    q, k, v = _tc_stage_a(x, pos_enc, params, BLK=1024)
    return _tc_stage_b(q, counts_parts, k, v, params, BLK=1024)


# speedup vs baseline: 1.1957x; 1.1957x over previous
import functools
import math

import jax
import jax.numpy as jnp
from jax import lax
from jax.experimental import pallas as pl
from jax.experimental.pallas import tpu as pltpu
from jax.experimental.pallas import tpu_sc as plsc


@functools.lru_cache(maxsize=None)
def _make_sc_hist(NN: int, C: int):
    L = 16
    NW = 32
    CH = (NN // NW) // L * L
    assert CH % 8 == 0
    TAIL = NN - CH * NW
    assert TAIL % L == 0 and (CH * NW) % 8 == 0

    mesh = plsc.VectorSubcoreMesh(core_axis_name="c", subcore_axis_name="s")

    @functools.partial(
        pl.kernel,
        mesh=mesh,
        out_type=jax.ShapeDtypeStruct((NW, C), jnp.int32),
        scratch_types=[
            pltpu.VMEM((CH,), jnp.int32),
            pltpu.VMEM((L * C,), jnp.int32),
            pltpu.VMEM((C,), jnp.int32),
            pltpu.VMEM((max(TAIL, L),), jnp.int32),
        ],
        compiler_params=pltpu.CompilerParams(needs_layout_passes=False),
    )
    def sc_hist(idx_hbm, out_hbm, idx_v, hist_v, part_v, tail_v):
        wid = lax.axis_index("s") * 2 + lax.axis_index("c")
        base = wid * CH
        pltpu.sync_copy(idx_hbm.at[pl.ds(base, CH)], idx_v)

        lane_base = lax.iota(jnp.int32, 16) * C
        ones = jnp.ones((L,), jnp.int32)
        zeros = jnp.zeros((L,), jnp.int32)

        def zbody(j, _):
            hist_v[pl.ds(j * L, L)] = zeros
            return _
        lax.fori_loop(0, (L * C) // L, zbody, 0, unroll=8)

        def sbody(i, _):
            vals = idx_v[pl.ds(i * L, L)]
            plsc.addupdate_scatter(hist_v, [lane_base + vals], ones)
            return _
        lax.fori_loop(0, CH // L, sbody, 0, unroll=8)

        if TAIL > 0:
            @pl.when(wid == 0)
            def _tail():
                pltpu.sync_copy(idx_hbm.at[pl.ds(CH * NW, TAIL)], tail_v)
                def tbody(i, _):
                    vals = tail_v[pl.ds(i * L, L)]
                    plsc.addupdate_scatter(hist_v, [lane_base + vals], ones)
                    return _
                lax.fori_loop(0, TAIL // L, tbody, 0, unroll=4)

        def rbody(j, _):
            acc = hist_v[pl.ds(j * L, L)]
            for l in range(1, L):
                acc = acc + hist_v[pl.ds(j * L + l * C, L)]
            part_v[pl.ds(j * L, L)] = acc
            return _
        lax.fori_loop(0, C // L, rbody, 0, unroll=4)

        pltpu.sync_copy(part_v, out_hbm.at[wid])

    return sc_hist


def _tc_a_body(x_ref, peT_ref,
               w1T_ref, b1_ref, w2_ref, b2_ref,
               proj_w_ref, proj_b_ref, qgT_ref, qg_b_ref,
               kgT_ref, kg_b_ref, vg_w_ref, vg_b_ref,
               vqk_ref, vqvT_ref,
               q_ref, k_ref, v_ref):
    i = pl.program_id(0)
    H = q_ref.shape[1]
    qscale = (1.0 / math.sqrt(H)) * math.log2(math.e)

    f32 = jnp.float32
    bf16 = jnp.bfloat16

    @pl.when(i == 0)
    def _kv():
        k = lax.dot_general(vqk_ref[...].astype(bf16), kgT_ref[...].astype(bf16),
                            (((1,), (1,)), ((), ())), preferred_element_type=f32)
        k_ref[...] = (k + kg_b_ref[...]).astype(bf16)
        v = lax.dot_general(vqvT_ref[...].astype(bf16), vg_w_ref[...].astype(bf16),
                            (((0,), (0,)), ((), ())), preferred_element_type=f32)
        v_ref[...] = (v + vg_b_ref[...]).astype(bf16)

    xb = x_ref[...].astype(bf16)
    h1 = lax.dot_general(xb, w1T_ref[...].astype(bf16),
                         (((1,), (1,)), ((), ())), preferred_element_type=f32)
    h1 = jnp.maximum(h1 + b1_ref[...], 0.0).astype(bf16)
    h = lax.dot_general(h1, w2_ref[...].astype(bf16),
                        (((1,), (0,)), ((), ())), preferred_element_type=f32)
    h = (h + b2_ref[...]).astype(bf16)
    p = lax.dot_general(h, proj_w_ref[...].astype(bf16),
                        (((1,), (0,)), ((), ())), preferred_element_type=f32)
    p = (p + proj_b_ref[...]).astype(bf16)
    GD = p.shape[1]
    q1 = lax.dot_general(p, qgT_ref[:, :GD].astype(bf16),
                         (((1,), (1,)), ((), ())), preferred_element_type=f32)
    q2 = lax.dot_general(peT_ref[...].astype(bf16), qgT_ref[:, GD:].astype(bf16),
                         (((0,), (1,)), ((), ())), preferred_element_type=f32)
    q = q1 + q2 + qg_b_ref[...]
    q_ref[...] = (q * qscale).astype(bf16)


def _tc_b_body(q_ref, cnt_ref, k_ref, v_ref,
               ff_w1_ref, ff_b1_ref, ff_w2_ref, ff_b2_ref,
               out_w_ref, out_b_ref,
               yT_ref, w2c_scr):
    i = pl.program_id(0)
    f32 = jnp.float32
    bf16 = jnp.bfloat16
    H = v_ref.shape[1]

    @pl.when(i == 0)
    def _fold_counts():
        cnt = jnp.sum(cnt_ref[...], axis=0).astype(f32)
        v = v_ref[...].astype(f32)
        w2c_scr[:, :H] = (v * cnt[:, None]).astype(bf16)
        w2c_scr[:, H:] = jnp.broadcast_to(cnt[:, None], (cnt.shape[0], H)).astype(bf16)

    dots = lax.dot_general(q_ref[...], k_ref[...], (((1,), (1,)), ((), ())),
                           preferred_element_type=f32)
    m = jnp.max(dots, axis=1, keepdims=True)
    e = jnp.exp2(dots - m).astype(bf16)
    nd = lax.dot_general(e, w2c_scr[...], (((1,), (0,)), ((), ())),
                         preferred_element_type=f32)
    out = nd[:, :H] / nd[:, H:H + 1]
    f = jnp.maximum(out @ ff_w1_ref[...] + ff_b1_ref[...], 0.0)
    f = jnp.maximum(f @ ff_w2_ref[...] + ff_b2_ref[...], 0.0)
    y = f @ out_w_ref[...] + out_b_ref[...]
    yT_ref[...] = y.T


def _row2(BLK, d):
    return pl.BlockSpec((BLK, d), lambda i: (i, 0))


def _col2(d, BLK):
    return pl.BlockSpec((d, BLK), lambda i: (0, i))


def _full(s):
    return pl.BlockSpec(s, lambda i: (0,) * len(s))


def _vec(d):
    return pl.BlockSpec((1, d), lambda i: (0, 0))


def _tc_stage_a(x, peT, params, BLK):
    B, IN = x.shape
    GD = peT.shape[0]
    C = params["vq_k"].shape[0]
    H = params["qg_b"].shape[0]
    grid = (B // BLK,)

    in_specs = [
        _row2(BLK, IN), _col2(GD, BLK),
        _full((H, IN)), _vec(H),
        _full((H, H)), _vec(H),
        _full((H, GD)), _vec(GD),
        _full((H, 2 * GD)), _vec(H),
        _full((H, 2 * GD)), _vec(H),
        _full((GD, H)), _vec(H),
        _full((C, 2 * GD)), _full((GD, C)),
    ]
    out_specs = [_row2(BLK, H), _full((C, H)), _full((C, H))]
    out_shape = [
        jax.ShapeDtypeStruct((B, H), jnp.bfloat16),
        jax.ShapeDtypeStruct((C, H), jnp.bfloat16),
        jax.ShapeDtypeStruct((C, H), jnp.bfloat16),
    ]
    args = [
        x, peT,
        params["fc_in_w1"].T, params["fc_in_b1"].reshape(1, -1),
        params["fc_in_w2"], params["fc_in_b2"].reshape(1, -1),
        params["proj_w"], params["proj_b"].reshape(1, -1),
        params["qg_w"].T, params["qg_b"].reshape(1, -1),
        params["kg_w"].T, params["kg_b"].reshape(1, -1),
        params["vg_w"], params["vg_b"].reshape(1, -1),
        params["vq_k"], params["vq_v"].T,
    ]
    return pl.pallas_call(
        _tc_a_body,
        grid=grid,
        in_specs=in_specs,
        out_specs=out_specs,
        out_shape=out_shape,
        compiler_params=pltpu.CompilerParams(
            dimension_semantics=("arbitrary",),
        ),
    )(*args)


def _tc_stage_b(q, counts_parts, k, v, params, BLK):
    B, H = q.shape
    C = k.shape[0]
    grid = (B // BLK,)

    in_specs = [
        _row2(BLK, H),
        _full(counts_parts.shape),
        _full((C, H)), _full((C, H)),
        _full((H, H)), _vec(H),
        _full((H, H)), _vec(H),
        _full((H, H)), _vec(H),
    ]
    args = [
        q, counts_parts, k, v,
        params["ff_w1"], params["ff_b1"].reshape(1, -1),
        params["ff_w2"], params["ff_b2"].reshape(1, -1),
        params["out_w"], params["out_b"].reshape(1, -1),
    ]
    yT = pl.pallas_call(
        _tc_b_body,
        grid=grid,
        in_specs=in_specs,
        out_specs=_col2(H, BLK),
        out_shape=jax.ShapeDtypeStruct((H, B), jnp.float32),
        scratch_shapes=[pltpu.VMEM((C, 2 * H), jnp.bfloat16)],
        compiler_params=pltpu.CompilerParams(
            dimension_semantics=("arbitrary",),
        ),
    )(*args)
    return yT.T


def kernel(seq, x, pos_enc, batch_idx, c_idx,
           fc_in_w1, fc_in_b1, fc_in_w2, fc_in_b2,
           fcs_w1, fcs_b1, fcs_w2, fcs_b2,
           proj_w, proj_b, qg_w, qg_b, kg_w, kg_b, vg_w, vg_b,
           vq_k, vq_v, ff_w1, ff_b1, ff_w2, ff_b2, out_w, out_b):
    NN = c_idx.shape[0]
    C = vq_k.shape[0]
    counts_parts = _make_sc_hist(NN, C)(c_idx.astype(jnp.int32))
    params = dict(
        fc_in_w1=fc_in_w1, fc_in_b1=fc_in_b1,
        fc_in_w2=fc_in_w2, fc_in_b2=fc_in_b2,
        proj_w=proj_w, proj_b=proj_b,
        qg_w=qg_w, qg_b=qg_b, kg_w=kg_w, kg_b=kg_b,
        vg_w=vg_w, vg_b=vg_b, vq_k=vq_k, vq_v=vq_v,
        ff_w1=ff_w1, ff_b1=ff_b1, ff_w2=ff_w2, ff_b2=ff_b2,
        out_w=out_w, out_b=out_b,
    )
    q, k, v = _tc_stage_a(x, pos_enc.T, params, BLK=1024)
    return _tc_stage_b(q, counts_parts, k, v, params, BLK=1024)


# speedup vs baseline: 1.3892x; 1.1618x over previous
import functools
import math

import jax
import jax.numpy as jnp
from jax import lax
from jax.experimental import pallas as pl
from jax.experimental.pallas import tpu as pltpu
from jax.experimental.pallas import tpu_sc as plsc


@functools.lru_cache(maxsize=None)
def _make_sc_hist(NN: int, C: int):
    L = 16
    NW = 32
    CH = (NN // NW) // L * L
    assert CH % 8 == 0
    TAIL = NN - CH * NW
    assert TAIL % L == 0 and (CH * NW) % 8 == 0

    mesh = plsc.VectorSubcoreMesh(core_axis_name="c", subcore_axis_name="s")

    NCHK = next((n for n in (3, 4, 2) if (CH // L) % n == 0), 1)
    CHNK = CH // NCHK
    assert CHNK % 8 == 0

    @functools.partial(
        pl.kernel,
        mesh=mesh,
        out_type=jax.ShapeDtypeStruct((NW, C), jnp.int32),
        scratch_types=[
            pltpu.VMEM((CHNK,), jnp.int32),
            pltpu.VMEM((CHNK,), jnp.int32),
            pltpu.VMEM((C,), jnp.int32),
            pltpu.VMEM((max(TAIL, L),), jnp.int32),
            pltpu.SemaphoreType.DMA,
            pltpu.SemaphoreType.DMA,
        ],
        compiler_params=pltpu.CompilerParams(needs_layout_passes=False),
    )
    def sc_hist(idx_hbm, out_hbm, buf0, buf1, hist_v, tail_v, sem0, sem1):
        wid = lax.axis_index("s") * 2 + lax.axis_index("c")
        base = wid * CH
        bufs = (buf0, buf1)
        sems = (sem0, sem1)

        ones = jnp.ones((L,), jnp.int32)
        zeros = jnp.zeros((L,), jnp.int32)

        handle = pltpu.async_copy(idx_hbm.at[pl.ds(base, CHNK)], buf0, sem0)

        def zbody(j, _):
            hist_v[pl.ds(j * L, L)] = zeros
            return _
        lax.fori_loop(0, C // L, zbody, 0, unroll=8)

        for c in range(NCHK):
            nxt = None
            if c + 1 < NCHK:
                nxt = pltpu.async_copy(
                    idx_hbm.at[pl.ds(base + (c + 1) * CHNK, CHNK)],
                    bufs[(c + 1) % 2], sems[(c + 1) % 2])
            handle.wait()
            buf = bufs[c % 2]

            def sbody(i, _):
                vals = buf[pl.ds(i * L, L)]
                plsc.addupdate_scatter(hist_v, [vals], ones)
                return _
            lax.fori_loop(0, CHNK // L, sbody, 0, unroll=8)
            handle = nxt

        if TAIL > 0:
            @pl.when(wid == 0)
            def _tail():
                pltpu.sync_copy(idx_hbm.at[pl.ds(CH * NW, TAIL)], tail_v)
                def tbody(i, _):
                    vals = tail_v[pl.ds(i * L, L)]
                    plsc.addupdate_scatter(hist_v, [vals], ones)
                    return _
                lax.fori_loop(0, TAIL // L, tbody, 0, unroll=4)

        pltpu.sync_copy(hist_v, out_hbm.at[wid])

    return sc_hist


def _tc_a_body(x_ref, peT_ref,
               w1T_ref, b1_ref, w2_ref, b2_ref,
               proj_w_ref, proj_b_ref, qgT_ref, qg_b_ref,
               kgT_ref, kg_b_ref, vg_w_ref, vg_b_ref,
               vqk_ref, vqvT_ref,
               q_ref, k_ref, v_ref):
    i = pl.program_id(0)
    H = q_ref.shape[1]
    qscale = (1.0 / math.sqrt(H)) * math.log2(math.e)

    f32 = jnp.float32
    bf16 = jnp.bfloat16

    @pl.when(i == 0)
    def _kv():
        k = lax.dot_general(vqk_ref[...].astype(bf16), kgT_ref[...].astype(bf16),
                            (((1,), (1,)), ((), ())), preferred_element_type=f32)
        k_ref[...] = (k + kg_b_ref[...]).astype(bf16)
        v = lax.dot_general(vqvT_ref[...].astype(bf16), vg_w_ref[...].astype(bf16),
                            (((0,), (0,)), ((), ())), preferred_element_type=f32)
        v_ref[...] = (v + vg_b_ref[...]).astype(bf16)

    xb = x_ref[...].astype(bf16)
    h1 = lax.dot_general(xb, w1T_ref[...].astype(bf16),
                         (((1,), (1,)), ((), ())), preferred_element_type=f32)
    h1 = jnp.maximum(h1 + b1_ref[...], 0.0).astype(bf16)
    h = lax.dot_general(h1, w2_ref[...].astype(bf16),
                        (((1,), (0,)), ((), ())), preferred_element_type=f32)
    h = (h + b2_ref[...]).astype(bf16)
    p = lax.dot_general(h, proj_w_ref[...].astype(bf16),
                        (((1,), (0,)), ((), ())), preferred_element_type=f32)
    p = (p + proj_b_ref[...]).astype(bf16)
    GD = p.shape[1]
    q1 = lax.dot_general(p, qgT_ref[:, :GD].astype(bf16),
                         (((1,), (1,)), ((), ())), preferred_element_type=f32)
    q2 = lax.dot_general(peT_ref[...].astype(bf16), qgT_ref[:, GD:].astype(bf16),
                         (((0,), (1,)), ((), ())), preferred_element_type=f32)
    q = q1 + q2 + qg_b_ref[...]
    q_ref[...] = (q * qscale).astype(bf16)


def _tc_b_body(q_ref, cnt_ref, k_ref, v_ref,
               ff_w1_ref, ff_b1_ref, ff_w2_ref, ff_b2_ref,
               out_w_ref, out_b_ref,
               yT_ref, w2c_scr):
    i = pl.program_id(0)
    f32 = jnp.float32
    bf16 = jnp.bfloat16
    H = v_ref.shape[1]

    @pl.when(i == 0)
    def _fold_counts():
        cnt = jnp.sum(cnt_ref[...], axis=0).astype(f32)
        v = v_ref[...].astype(f32)
        w2c_scr[:, :H] = (v * cnt[:, None]).astype(bf16)
        w2c_scr[:, H:] = jnp.broadcast_to(cnt[:, None], (cnt.shape[0], H)).astype(bf16)

    dots = lax.dot_general(q_ref[...], k_ref[...], (((1,), (1,)), ((), ())),
                           preferred_element_type=f32)
    m = jnp.max(dots, axis=1, keepdims=True)
    e = jnp.exp2(dots - m).astype(bf16)
    nd = lax.dot_general(e, w2c_scr[...], (((1,), (0,)), ((), ())),
                         preferred_element_type=f32)
    out = nd[:, :H] / nd[:, H:H + 1]
    f = jnp.maximum(out @ ff_w1_ref[...] + ff_b1_ref[...], 0.0)
    f = jnp.maximum(f @ ff_w2_ref[...] + ff_b2_ref[...], 0.0)
    y = f @ out_w_ref[...] + out_b_ref[...]
    yT_ref[...] = y.T


def _row2(BLK, d):
    return pl.BlockSpec((BLK, d), lambda i: (i, 0))


def _col2(d, BLK):
    return pl.BlockSpec((d, BLK), lambda i: (0, i))


def _full(s):
    return pl.BlockSpec(s, lambda i: (0,) * len(s))


def _vec(d):
    return pl.BlockSpec((1, d), lambda i: (0, 0))


def _tc_stage_a(x, peT, params, BLK):
    B, IN = x.shape
    GD = peT.shape[0]
    C = params["vq_k"].shape[0]
    H = params["qg_b"].shape[0]
    grid = (B // BLK,)

    in_specs = [
        _row2(BLK, IN), _col2(GD, BLK),
        _full((H, IN)), _vec(H),
        _full((H, H)), _vec(H),
        _full((H, GD)), _vec(GD),
        _full((H, 2 * GD)), _vec(H),
        _full((H, 2 * GD)), _vec(H),
        _full((GD, H)), _vec(H),
        _full((C, 2 * GD)), _full((GD, C)),
    ]
    out_specs = [_row2(BLK, H), _full((C, H)), _full((C, H))]
    out_shape = [
        jax.ShapeDtypeStruct((B, H), jnp.bfloat16),
        jax.ShapeDtypeStruct((C, H), jnp.bfloat16),
        jax.ShapeDtypeStruct((C, H), jnp.bfloat16),
    ]
    args = [
        x, peT,
        params["fc_in_w1"].T, params["fc_in_b1"].reshape(1, -1),
        params["fc_in_w2"], params["fc_in_b2"].reshape(1, -1),
        params["proj_w"], params["proj_b"].reshape(1, -1),
        params["qg_w"].T, params["qg_b"].reshape(1, -1),
        params["kg_w"].T, params["kg_b"].reshape(1, -1),
        params["vg_w"], params["vg_b"].reshape(1, -1),
        params["vq_k"], params["vq_v"].T,
    ]
    return pl.pallas_call(
        _tc_a_body,
        grid=grid,
        in_specs=in_specs,
        out_specs=out_specs,
        out_shape=out_shape,
        compiler_params=pltpu.CompilerParams(
            dimension_semantics=("arbitrary",),
        ),
    )(*args)


def _tc_stage_b(q, counts_parts, k, v, params, BLK):
    B, H = q.shape
    C = k.shape[0]
    grid = (B // BLK,)

    in_specs = [
        _row2(BLK, H),
        _full(counts_parts.shape),
        _full((C, H)), _full((C, H)),
        _full((H, H)), _vec(H),
        _full((H, H)), _vec(H),
        _full((H, H)), _vec(H),
    ]
    args = [
        q, counts_parts, k, v,
        params["ff_w1"], params["ff_b1"].reshape(1, -1),
        params["ff_w2"], params["ff_b2"].reshape(1, -1),
        params["out_w"], params["out_b"].reshape(1, -1),
    ]
    yT = pl.pallas_call(
        _tc_b_body,
        grid=grid,
        in_specs=in_specs,
        out_specs=_col2(H, BLK),
        out_shape=jax.ShapeDtypeStruct((H, B), jnp.float32),
        scratch_shapes=[pltpu.VMEM((C, 2 * H), jnp.bfloat16)],
        compiler_params=pltpu.CompilerParams(
            dimension_semantics=("arbitrary",),
        ),
    )(*args)
    return yT.T


def kernel(seq, x, pos_enc, batch_idx, c_idx,
           fc_in_w1, fc_in_b1, fc_in_w2, fc_in_b2,
           fcs_w1, fcs_b1, fcs_w2, fcs_b2,
           proj_w, proj_b, qg_w, qg_b, kg_w, kg_b, vg_w, vg_b,
           vq_k, vq_v, ff_w1, ff_b1, ff_w2, ff_b2, out_w, out_b):
    NN = c_idx.shape[0]
    C = vq_k.shape[0]
    counts_parts = _make_sc_hist(NN, C)(c_idx.astype(jnp.int32))
    params = dict(
        fc_in_w1=fc_in_w1, fc_in_b1=fc_in_b1,
        fc_in_w2=fc_in_w2, fc_in_b2=fc_in_b2,
        proj_w=proj_w, proj_b=proj_b,
        qg_w=qg_w, qg_b=qg_b, kg_w=kg_w, kg_b=kg_b,
        vg_w=vg_w, vg_b=vg_b, vq_k=vq_k, vq_v=vq_v,
        ff_w1=ff_w1, ff_b1=ff_b1, ff_w2=ff_w2, ff_b2=ff_b2,
        out_w=out_w, out_b=out_b,
    )
    q, k, v = _tc_stage_a(x, pos_enc.T, params, BLK=4096)
    return _tc_stage_b(q, counts_parts, k, v, params, BLK=8192)
